# baseline (speedup 1.0000x reference)
import jax
import jax.numpy as jnp
from jax.experimental import pallas as pl

_B_BLK = 64


def _add_body(x_ref, pos_ref, o_ref):
    o_ref[...] = x_ref[...] + pos_ref[...]


def kernel(x, pos_table):
    batch, maxlen, embed = x.shape
    flat = maxlen * embed
    x2 = x.reshape(batch, flat)
    pos2 = pos_table.reshape(1, flat)

    grid = (batch // _B_BLK,)
    out = pl.pallas_call(
        _add_body,
        grid=grid,
        in_specs=[
            pl.BlockSpec((_B_BLK, flat), lambda i: (i, 0)),
            pl.BlockSpec((1, flat), lambda i: (0, 0)),
        ],
        out_specs=pl.BlockSpec((_B_BLK, flat), lambda i: (i, 0)),
        out_shape=jax.ShapeDtypeStruct((batch, flat), x.dtype),
    )(x2, pos2)
    return out.reshape(batch, maxlen, embed)


# speedup vs baseline: 3.2023x; 3.2023x over previous
import jax
import jax.numpy as jnp
from jax.experimental import pallas as pl

_F_BLK = 128


def _add_body(x_ref, pos_ref, o_ref):
    o_ref[...] = x_ref[...] + pos_ref[...]


def kernel(x, pos_table):
    batch, maxlen, embed = x.shape
    flat = maxlen * embed
    xt = x.transpose(1, 2, 0).reshape(flat, batch)
    post = pos_table.reshape(flat, 1)

    grid = (flat // _F_BLK,)
    out_t = pl.pallas_call(
        _add_body,
        grid=grid,
        in_specs=[
            pl.BlockSpec((_F_BLK, batch), lambda i: (i, 0)),
            pl.BlockSpec((_F_BLK, 1), lambda i: (i, 0)),
        ],
        out_specs=pl.BlockSpec((_F_BLK, batch), lambda i: (i, 0)),
        out_shape=jax.ShapeDtypeStruct((flat, batch), x.dtype),
    )(xt, post)
    return out_t.reshape(maxlen, embed, batch).transpose(2, 0, 1)


# speedup vs baseline: 3.5434x; 1.1065x over previous
import jax
import jax.numpy as jnp
from jax.experimental import pallas as pl

_F_BLK = 512


def _add_body(x_ref, pos_ref, o_ref):
    o_ref[...] = x_ref[...] + pos_ref[...]


def kernel(x, pos_table):
    batch, maxlen, embed = x.shape
    flat = maxlen * embed
    xt = x.transpose(1, 2, 0).reshape(flat, batch)
    post = pos_table.reshape(flat, 1)

    grid = (flat // _F_BLK,)
    out_t = pl.pallas_call(
        _add_body,
        grid=grid,
        in_specs=[
            pl.BlockSpec((_F_BLK, batch), lambda i: (i, 0)),
            pl.BlockSpec((_F_BLK, 1), lambda i: (i, 0)),
        ],
        out_specs=pl.BlockSpec((_F_BLK, batch), lambda i: (i, 0)),
        out_shape=jax.ShapeDtypeStruct((flat, batch), x.dtype),
    )(xt, post)
    return out_t.reshape(maxlen, embed, batch).transpose(2, 0, 1)
